# Initial kernel scaffold; baseline (speedup 1.0000x reference)
import jax
import jax.numpy as jnp
from jax.experimental import pallas as pl


def kernel(x, Wg, W1, W2):
    raise NotImplementedError("write your pallas kernel here")


# speedup vs baseline: 1.9059x; 1.9059x over previous
import functools

import jax
import jax.numpy as jnp
from jax.experimental import pallas as pl
from jax.experimental.pallas import tpu as pltpu


def _router_kernel(x_ref, wg_ref, logits_ref, probs_ref, *, n_experts):
    x = x_ref[...]
    wg = wg_ref[...]
    logits = jax.lax.dot_general(
        x, wg, (((1,), (1,)), ((), ())), preferred_element_type=jnp.float32)
    logits_ref[...] = logits
    lane = jax.lax.broadcasted_iota(jnp.int32, logits.shape, 1)
    masked = jnp.where(lane < n_experts, logits, -jnp.inf)
    m = jnp.max(masked, axis=1, keepdims=True)
    ex = jnp.where(lane < n_experts, jnp.exp(masked - m), 0.0)
    probs_ref[...] = ex / jnp.sum(ex, axis=1, keepdims=True)


def _ffn_kernel(te_ref, xg_ref, w1_ref, w2_ref, w_ref, out_ref):
    del te_ref
    xg = xg_ref[...]
    w1 = w1_ref[0]
    w2 = w2_ref[0]
    h = jax.lax.dot_general(
        xg, w1, (((1,), (1,)), ((), ())), preferred_element_type=jnp.float32)
    h = 0.5 * h * (1.0 + jax.lax.erf(h * (2.0 ** -0.5)))
    o = jax.lax.dot_general(
        h, w2, (((1,), (1,)), ((), ())), preferred_element_type=jnp.float32)
    out_ref[...] = o * w_ref[...][:, 0:1]


def kernel(x, Wg, W1, W2):
    B_, S_, H_ = x.shape
    E_, I_, _ = W1.shape
    N = B_ * S_
    x2 = x.reshape(N, H_)

    EP = 128
    wg_pad = jnp.zeros((EP, H_), jnp.float32).at[:E_].set(Wg)
    logits_pad, probs_pad = pl.pallas_call(
        functools.partial(_router_kernel, n_experts=E_),
        out_shape=(
            jax.ShapeDtypeStruct((N, EP), jnp.float32),
            jax.ShapeDtypeStruct((N, EP), jnp.float32),
        ),
    )(x2, wg_pad)
    logits = logits_pad[:, :E_]
    probs = probs_pad[:, :E_]

    cap = max(1, N // E_)
    aff = probs.T
    scores, idxs = jax.lax.top_k(aff, cap)
    sel = jnp.full((E_, N), -jnp.inf, probs.dtype)
    sel = sel.at[jnp.arange(E_)[:, None], idxs].set(scores)
    best_score = jnp.max(sel, axis=0)
    best_expert = jnp.argmax(sel, axis=0)
    assigned = best_score > -jnp.inf
    fallback = jnp.argmax(probs, axis=1)
    expert_idx = jnp.where(assigned, best_expert, fallback)
    weight = jnp.take_along_axis(probs, expert_idx[:, None], axis=1)[:, 0]

    T = 128
    num_tiles = N // T + E_
    S_slots = num_tiles * T
    order = jnp.argsort(expert_idx)
    c = jnp.bincount(expert_idx, length=E_)
    coff = jnp.concatenate([jnp.zeros((1,), c.dtype), jnp.cumsum(c)[:-1]])
    pc = ((c + T - 1) // T) * T
    pend = jnp.cumsum(pc)
    poff = pend - pc
    slot = jnp.arange(S_slots)
    e_of = jnp.searchsorted(pend, slot, side="right")
    e_c = jnp.minimum(e_of, E_ - 1)
    r = slot - poff[e_c]
    valid = (e_of < E_) & (r < c[e_c])
    src = jnp.where(valid, coff[e_c] + r, 0)
    tid = jnp.where(valid, order[src], 0)
    w_slot = jnp.where(valid, weight[tid], 0.0)
    tile_expert = e_c.reshape(num_tiles, T)[:, 0].astype(jnp.int32)

    xg = x2[tid]
    w_in = jnp.broadcast_to(w_slot[:, None], (S_slots, 128))

    grid_spec = pltpu.PrefetchScalarGridSpec(
        num_scalar_prefetch=1,
        grid=(num_tiles,),
        in_specs=[
            pl.BlockSpec((T, H_), lambda i, te: (i, 0)),
            pl.BlockSpec((1, I_, H_), lambda i, te: (te[i], 0, 0)),
            pl.BlockSpec((1, H_, I_), lambda i, te: (te[i], 0, 0)),
            pl.BlockSpec((T, 128), lambda i, te: (i, 0)),
        ],
        out_specs=pl.BlockSpec((T, H_), lambda i, te: (i, 0)),
    )
    out_slots = pl.pallas_call(
        _ffn_kernel,
        grid_spec=grid_spec,
        out_shape=jax.ShapeDtypeStruct((S_slots, H_), jnp.float32),
    )(tile_expert, xg, W1, W2, w_in)

    out2 = jnp.zeros((N, H_), jnp.float32).at[tid].add(out_slots)

    return (out2.reshape(B_, S_, H_),
            weight.reshape(B_, S_),
            expert_idx.reshape(B_, S_),
            logits,
            probs)


# speedup vs baseline: 3.6164x; 1.8974x over previous
import functools

import jax
import jax.numpy as jnp
from jax.experimental import pallas as pl
from jax.experimental.pallas import tpu as pltpu


def _router_kernel(x_ref, wg_ref, logits_ref, probs_ref, *, n_experts):
    x = x_ref[...]
    wg = wg_ref[...]
    logits = jax.lax.dot_general(
        x, wg, (((1,), (1,)), ((), ())), preferred_element_type=jnp.float32)
    logits_ref[...] = logits
    lane = jax.lax.broadcasted_iota(jnp.int32, logits.shape, 1)
    masked = jnp.where(lane < n_experts, logits, -jnp.inf)
    m = jnp.max(masked, axis=1, keepdims=True)
    ex = jnp.where(lane < n_experts, jnp.exp(masked - m), 0.0)
    probs_ref[...] = ex / jnp.sum(ex, axis=1, keepdims=True)


def _ffn_kernel(te_ref, sot_ref, w_ref, x_ref, w1_ref, w2_ref, out_ref, *, tile):
    del te_ref
    i = pl.program_id(0)
    sot = sot_ref[...]
    n = sot.shape[1]
    slot_iota = jax.lax.broadcasted_iota(jnp.int32, (tile, n), 0) + i * tile
    g = (slot_iota == sot).astype(jnp.float32)
    xg = jax.lax.dot_general(
        g, x_ref[...], (((1,), (0,)), ((), ())),
        preferred_element_type=jnp.float32)
    w1 = w1_ref[0]
    w2 = w2_ref[0]
    h = jax.lax.dot_general(
        xg, w1, (((1,), (1,)), ((), ())), preferred_element_type=jnp.float32)
    h = 0.5 * h * (1.0 + jax.lax.erf(h * (2.0 ** -0.5)))
    o = jax.lax.dot_general(
        h, w2, (((1,), (1,)), ((), ())), preferred_element_type=jnp.float32)
    wslot = jnp.max(g * w_ref[...], axis=1, keepdims=True)
    out_ref[...] = o * wslot


def _scatter_kernel(sot_ref, os_ref, out_ref, *, tile, n_slots):
    s_col = sot_ref[...][:, 0:1]
    iota = jax.lax.broadcasted_iota(jnp.int32, (tile, n_slots), 1)
    g = (iota == s_col).astype(jnp.float32)
    out_ref[...] = jax.lax.dot_general(
        g, os_ref[...], (((1,), (0,)), ((), ())),
        preferred_element_type=jnp.float32)


def kernel(x, Wg, W1, W2):
    B_, S_, H_ = x.shape
    E_, I_, _ = W1.shape
    N = B_ * S_
    x2 = x.reshape(N, H_)

    EP = 128
    wg_pad = jnp.zeros((EP, H_), jnp.float32).at[:E_].set(Wg)
    logits_pad, probs_pad = pl.pallas_call(
        functools.partial(_router_kernel, n_experts=E_),
        out_shape=(
            jax.ShapeDtypeStruct((N, EP), jnp.float32),
            jax.ShapeDtypeStruct((N, EP), jnp.float32),
        ),
    )(x2, wg_pad)
    logits = logits_pad[:, :E_]
    probs = probs_pad[:, :E_]

    cap = max(1, N // E_)
    aff = probs.T
    scores, idxs = jax.lax.top_k(aff, cap)
    sel = jnp.full((E_, N), -jnp.inf, probs.dtype)
    sel = sel.at[jnp.arange(E_)[:, None], idxs].set(scores)
    best_score = jnp.max(sel, axis=0)
    best_expert = jnp.argmax(sel, axis=0)
    assigned = best_score > -jnp.inf
    fallback = jnp.argmax(probs, axis=1)
    expert_idx = jnp.where(assigned, best_expert, fallback)
    weight = jnp.take_along_axis(probs, expert_idx[:, None], axis=1)[:, 0]

    T = 128
    num_tiles = N // T + E_
    S_slots = num_tiles * T
    oh = (expert_idx[:, None] == jnp.arange(E_)[None, :]).astype(jnp.int32)
    rank = jnp.take_along_axis(jnp.cumsum(oh, axis=0) - 1,
                               expert_idx[:, None], axis=1)[:, 0]
    c = jnp.sum(oh, axis=0)
    pc = ((c + T - 1) // T) * T
    pend = jnp.cumsum(pc)
    poff = pend - pc
    sot = (poff[expert_idx] + rank).astype(jnp.int32)
    tile_id = jnp.arange(num_tiles)
    tile_expert = jnp.minimum(
        jnp.searchsorted(pend, tile_id * T, side="right"),
        E_ - 1).astype(jnp.int32)

    grid_spec = pltpu.PrefetchScalarGridSpec(
        num_scalar_prefetch=1,
        grid=(num_tiles,),
        in_specs=[
            pl.BlockSpec((1, N), lambda i, te: (0, 0)),
            pl.BlockSpec((1, N), lambda i, te: (0, 0)),
            pl.BlockSpec((N, H_), lambda i, te: (0, 0)),
            pl.BlockSpec((1, I_, H_), lambda i, te: (te[i], 0, 0)),
            pl.BlockSpec((1, H_, I_), lambda i, te: (te[i], 0, 0)),
        ],
        out_specs=pl.BlockSpec((T, H_), lambda i, te: (i, 0)),
    )
    o_slots = pl.pallas_call(
        functools.partial(_ffn_kernel, tile=T),
        grid_spec=grid_spec,
        out_shape=jax.ShapeDtypeStruct((S_slots, H_), jnp.float32),
    )(tile_expert, sot[None, :], weight[None, :], x2, W1, W2)

    sot_rep = jnp.broadcast_to(sot[:, None], (N, 128))
    out2 = pl.pallas_call(
        functools.partial(_scatter_kernel, tile=T, n_slots=S_slots),
        grid=(N // T,),
        in_specs=[
            pl.BlockSpec((T, 128), lambda i: (i, 0)),
            pl.BlockSpec((S_slots, H_), lambda i: (0, 0)),
        ],
        out_specs=pl.BlockSpec((T, H_), lambda i: (i, 0)),
        out_shape=jax.ShapeDtypeStruct((N, H_), jnp.float32),
    )(sot_rep, o_slots)

    return (out2.reshape(B_, S_, H_),
            weight.reshape(B_, S_),
            expert_idx.reshape(B_, S_),
            logits,
            probs)
